# baseline (speedup 1.0000x reference)
import jax
import jax.numpy as jnp
from jax import lax
from jax.experimental import pallas as pl
from jax.experimental.pallas import tpu as pltpu
from jax.experimental.pallas import tpu_sc as plsc

NC, NS = 2, 16
NW = NC * NS
CW = 1664
TB = 100096


def _stream_chunks(src, dst, nchunks, wid, bufs, isems, osems):
    def off(c):
        return pl.multiple_of(c * CW, 128)

    def in_copy(c, b):
        return pltpu.make_async_copy(
            src.at[:, pl.ds(off(c), CW)], bufs[b], isems[b]
        )

    def out_copy(c, b):
        return pltpu.make_async_copy(
            bufs[b], dst.at[:, pl.ds(off(c), CW)], osems[b]
        )

    maxk = -(-nchunks // NW)

    def c_of(k):
        return wid + k * NW

    @pl.when(c_of(0) < nchunks)
    def _():
        in_copy(c_of(0), 0).start()

    if maxk > 1:
        @pl.when(c_of(1) < nchunks)
        def _():
            in_copy(c_of(1), 1).start()

    mk2 = (maxk // 2) * 2

    @pl.loop(0, mk2, step=2)
    def _(k):
        for b in (0, 1):
            c = c_of(k + b)

            @pl.when(c < nchunks)
            def _():
                in_copy(c, b).wait()
                out_copy(c, b).start()
                nc = c + 2 * NW

                @pl.when(nc < nchunks)
                def _():
                    out_copy(c, b).wait()
                    in_copy(nc, b).start()

    if maxk % 2:
        b = (maxk - 1) % 2
        c = c_of(maxk - 1)

        @pl.when(c < nchunks)
        def _():
            in_copy(c, b).wait()
            out_copy(c, b).start()

    for b in (0, 1):
        if b < maxk:
            @pl.when(c_of(b) < nchunks)
            def _():
                out_copy(0, b).wait()


def _rag_copy(src, dst, buf, ncols):
    full = (ncols // CW) * CW
    rem = ((ncols - full) // 128) * 128
    if rem:
        pltpu.sync_copy(src.at[:, pl.ds(full, rem)], buf.at[:, pl.ds(0, rem)])
        pltpu.sync_copy(buf.at[:, pl.ds(0, rem)], dst.at[:, pl.ds(full, rem)])


def _sc_body(h_in, r_in, m_in, h_out, r_out, m_out,
             buf0, buf1, rbuf, mbuf, is0, is1, os0, os1):
    wid = lax.axis_index("c") * NS + lax.axis_index("s")
    bufs, isems, osems = (buf0, buf1), (is0, is1), (os0, os1)

    h_cols = h_in.shape[1]
    _stream_chunks(h_in, h_out, h_cols // CW, wid, bufs, isems, osems)

    @pl.when(wid == 24)
    def _():
        _rag_copy(h_in, h_out, buf1, h_cols)

    @pl.when(wid == 0)
    def _():
        pltpu.sync_copy(r_in, rbuf)
        pltpu.sync_copy(rbuf, r_out)

    @pl.when(wid == NS)
    def _():
        pltpu.sync_copy(m_in, mbuf)
        pltpu.sync_copy(mbuf, m_out)


def _tc_body(t_in, t_out):
    t_out[...] = t_in[...]


def kernel(head_w, relation_w, tail_w, r_mat):
    h_t, t_t = head_w.T, tail_w.T

    sc_run = pl.kernel(
        _sc_body,
        out_type=(
            jax.ShapeDtypeStruct(h_t.shape, h_t.dtype),
            jax.ShapeDtypeStruct(relation_w.shape, relation_w.dtype),
            jax.ShapeDtypeStruct(r_mat.shape, r_mat.dtype),
        ),
        mesh=plsc.VectorSubcoreMesh(core_axis_name="c", subcore_axis_name="s"),
        scratch_types=[
            pltpu.VMEM((32, CW), jnp.float32),
            pltpu.VMEM((32, CW), jnp.float32),
            pltpu.VMEM((4, 32), jnp.float32),
            pltpu.VMEM((4, 32, 32), jnp.float32),
            pltpu.SemaphoreType.DMA,
            pltpu.SemaphoreType.DMA,
            pltpu.SemaphoreType.DMA,
            pltpu.SemaphoreType.DMA,
        ],
    )
    grid = -(-t_t.shape[1] // TB)
    t_o = pl.pallas_call(
        _tc_body,
        grid=(grid,),
        in_specs=[pl.BlockSpec((32, TB), lambda i: (0, i))],
        out_specs=pl.BlockSpec((32, TB), lambda i: (0, i)),
        out_shape=jax.ShapeDtypeStruct(t_t.shape, t_t.dtype),
        compiler_params=pltpu.CompilerParams(
            dimension_semantics=("arbitrary",),
        ),
    )(t_t)

    h_o, r_o, m_o = sc_run(h_t, relation_w, r_mat)

    ncols = h_t.shape[1]
    done = (ncols // 128) * 128
    sliver = lax.slice(h_t, (0, done), h_t.shape)
    h_o = lax.dynamic_update_slice(h_o, sliver, (0, done))

    return h_o.T, r_o, t_o.T, m_o


# speedup vs baseline: 1.0024x; 1.0024x over previous
import jax
import jax.numpy as jnp
from jax import lax
from jax.experimental import pallas as pl
from jax.experimental.pallas import tpu as pltpu
from jax.experimental.pallas import tpu_sc as plsc

NC, NS = 2, 16
NW = NC * NS
CW = 1664
TB = 114688


def _stream_chunks(src, dst, nchunks, wid, bufs, isems, osems):
    def off(c):
        return pl.multiple_of(c * CW, 128)

    def in_copy(c, b):
        return pltpu.make_async_copy(
            src.at[:, pl.ds(off(c), CW)], bufs[b], isems[b]
        )

    def out_copy(c, b):
        return pltpu.make_async_copy(
            bufs[b], dst.at[:, pl.ds(off(c), CW)], osems[b]
        )

    maxk = -(-nchunks // NW)

    def c_of(k):
        return wid + k * NW

    @pl.when(c_of(0) < nchunks)
    def _():
        in_copy(c_of(0), 0).start()

    if maxk > 1:
        @pl.when(c_of(1) < nchunks)
        def _():
            in_copy(c_of(1), 1).start()

    mk2 = (maxk // 2) * 2

    @pl.loop(0, mk2, step=2)
    def _(k):
        for b in (0, 1):
            c = c_of(k + b)

            @pl.when(c < nchunks)
            def _():
                in_copy(c, b).wait()
                out_copy(c, b).start()
                nc = c + 2 * NW

                @pl.when(nc < nchunks)
                def _():
                    out_copy(c, b).wait()
                    in_copy(nc, b).start()

    if maxk % 2:
        b = (maxk - 1) % 2
        c = c_of(maxk - 1)

        @pl.when(c < nchunks)
        def _():
            in_copy(c, b).wait()
            out_copy(c, b).start()

    for b in (0, 1):
        if b < maxk:
            @pl.when(c_of(b) < nchunks)
            def _():
                out_copy(0, b).wait()


def _rag_copy(src, dst, buf, ncols):
    full = (ncols // CW) * CW
    rem = ((ncols - full) // 128) * 128
    if rem:
        pltpu.sync_copy(src.at[:, pl.ds(full, rem)], buf.at[:, pl.ds(0, rem)])
        pltpu.sync_copy(buf.at[:, pl.ds(0, rem)], dst.at[:, pl.ds(full, rem)])


def _sc_body(h_in, r_in, m_in, h_out, r_out, m_out,
             buf0, buf1, rbuf, mbuf, is0, is1, os0, os1):
    wid = lax.axis_index("c") * NS + lax.axis_index("s")
    bufs, isems, osems = (buf0, buf1), (is0, is1), (os0, os1)

    h_cols = h_in.shape[1]
    _stream_chunks(h_in, h_out, h_cols // CW, wid, bufs, isems, osems)

    @pl.when(wid == 24)
    def _():
        _rag_copy(h_in, h_out, buf1, h_cols)

    @pl.when(wid == 0)
    def _():
        pltpu.sync_copy(r_in, rbuf)
        pltpu.sync_copy(rbuf, r_out)

    @pl.when(wid == NS)
    def _():
        pltpu.sync_copy(m_in, mbuf)
        pltpu.sync_copy(mbuf, m_out)


def _tc_body(t_in, t_out):
    t_out[...] = t_in[...]


def kernel(head_w, relation_w, tail_w, r_mat):
    h_t, t_t = head_w.T, tail_w.T

    sc_run = pl.kernel(
        _sc_body,
        out_type=(
            jax.ShapeDtypeStruct(h_t.shape, h_t.dtype),
            jax.ShapeDtypeStruct(relation_w.shape, relation_w.dtype),
            jax.ShapeDtypeStruct(r_mat.shape, r_mat.dtype),
        ),
        mesh=plsc.VectorSubcoreMesh(core_axis_name="c", subcore_axis_name="s"),
        scratch_types=[
            pltpu.VMEM((32, CW), jnp.float32),
            pltpu.VMEM((32, CW), jnp.float32),
            pltpu.VMEM((4, 32), jnp.float32),
            pltpu.VMEM((4, 32, 32), jnp.float32),
            pltpu.SemaphoreType.DMA,
            pltpu.SemaphoreType.DMA,
            pltpu.SemaphoreType.DMA,
            pltpu.SemaphoreType.DMA,
        ],
    )
    grid = -(-t_t.shape[1] // TB)
    t_o = pl.pallas_call(
        _tc_body,
        grid=(grid,),
        in_specs=[pl.BlockSpec((32, TB), lambda i: (0, i))],
        out_specs=pl.BlockSpec((32, TB), lambda i: (0, i)),
        out_shape=jax.ShapeDtypeStruct(t_t.shape, t_t.dtype),
        compiler_params=pltpu.CompilerParams(
            dimension_semantics=("arbitrary",),
        ),
    )(t_t)

    h_o, r_o, m_o = sc_run(h_t, relation_w, r_mat)

    ncols = h_t.shape[1]
    done = (ncols // 128) * 128
    sliver = lax.slice(h_t, (0, done), h_t.shape)
    h_o = lax.dynamic_update_slice(h_o, sliver, (0, done))

    return h_o.T, r_o, t_o.T, m_o
